# baseline (speedup 1.0000x reference)
import functools

import jax
import jax.numpy as jnp
import numpy as np
from jax.experimental import pallas as pl

D_MODEL = 768
N_HEADS = 12
D_K = D_MODEL // N_HEADS
D_FF = 768
NUM_EXPERTS = 8
TOP_K = 2
T = 2048
EPS = float(jnp.finfo(jnp.float32).eps)

ROW_BLK = 256
HIGHEST = jax.lax.Precision.HIGHEST


def _rope_rotate(v):
    left = jnp.roll(v, -1, axis=-1)
    right = jnp.roll(v, 1, axis=-1)
    lane = jax.lax.broadcasted_iota(jnp.int32, v.shape, 1)
    even = (lane % 2) == 0
    return jnp.where(even, -left, right)


def _qkv_kernel(x_ref, n1_ref, w_ref, qnw_ref, knw_ref, cos_ref, sin_ref,
                g_ref, gt_ref, q_ref, k_ref, v_ref):
    xr = x_ref[...]
    ms = jnp.mean(xr * xr, axis=-1, keepdims=True)
    xn = xr / jnp.sqrt(ms + EPS) * n1_ref[...]
    qkv = jax.lax.dot(xn, w_ref[...], precision=HIGHEST)
    q = qkv[:, :D_MODEL]
    k = qkv[:, D_MODEL:2 * D_MODEL]
    v = qkv[:, 2 * D_MODEL:]

    cos = cos_ref[...]
    sin = sin_ref[...]
    g = g_ref[...]
    gt = gt_ref[...]

    def headnorm_rope(a, w_head):
        ss = jax.lax.dot(a * a, g, precision=HIGHEST) * (1.0 / D_K)
        inv = 1.0 / jnp.sqrt(ss + EPS)
        invf = jax.lax.dot(inv, gt, precision=HIGHEST)
        an = a * invf * w_head
        return an * cos + _rope_rotate(an) * sin

    qo = headnorm_rope(q, qnw_ref[...])
    ko = headnorm_rope(k, knw_ref[...])
    for h in range(N_HEADS):
        sl = slice(h * D_K, (h + 1) * D_K)
        q_ref[h] = qo[:, sl]
        k_ref[h] = ko[:, sl]
        v_ref[h] = v[:, sl]


def _attn_kernel(q_ref, k_ref, v_ref, o_ref):
    qb = pl.program_id(1)
    q = q_ref[0]
    k = k_ref[0]
    s = jax.lax.dot_general(q, k, (((1,), (1,)), ((), ())),
                            precision=HIGHEST) * (1.0 / float(np.sqrt(D_K)))
    row = qb * ROW_BLK + jax.lax.broadcasted_iota(jnp.int32, s.shape, 0)
    col = jax.lax.broadcasted_iota(jnp.int32, s.shape, 1)
    s = jnp.where(row >= col, s, jnp.float32(-1e30))
    m = jnp.max(s, axis=-1, keepdims=True)
    e = jnp.exp(s - m)
    p = e / jnp.sum(e, axis=-1, keepdims=True)
    o_ref[0] = jax.lax.dot(p, v_ref[0], precision=HIGHEST)


def _proj_router_kernel(x_ref, a_ref, wo_ref, n2_ref, gw_ref,
                        o1_ref, xn_ref, gates_ref, imp_ref, cnt_ref):
    i = pl.program_id(0)
    a = jnp.concatenate([a_ref[h] for h in range(N_HEADS)], axis=-1)
    o1 = x_ref[...] + jax.lax.dot(a, wo_ref[...], precision=HIGHEST)
    o1_ref[...] = o1
    ms = jnp.mean(o1 * o1, axis=-1, keepdims=True)
    xn = o1 / jnp.sqrt(ms + EPS) * n2_ref[...]
    xn_ref[...] = xn

    logits = jax.lax.dot(xn, gw_ref[...], precision=HIGHEST)
    lm = jnp.max(logits, axis=-1, keepdims=True)
    el = jnp.exp(logits - lm)
    probs = el / jnp.sum(el, axis=-1, keepdims=True)

    lane = jax.lax.broadcasted_iota(jnp.int32, probs.shape, 1)
    m1 = jnp.max(probs, axis=-1, keepdims=True)
    i1 = jnp.min(jnp.where(probs == m1, lane, NUM_EXPERTS),
                 axis=-1, keepdims=True)
    masked = jnp.where(lane == i1, jnp.float32(-1.0), probs)
    m2 = jnp.max(masked, axis=-1, keepdims=True)
    i2 = jnp.min(jnp.where(masked == m2, lane, NUM_EXPERTS),
                 axis=-1, keepdims=True)
    denom = m1 + m2
    g1 = m1 / denom
    g2 = m2 / denom
    gates = (jnp.where(lane == i1, g1, 0.0)
             + jnp.where(lane == i2, g2, 0.0))
    gates_ref[...] = gates

    @pl.when(i == 0)
    def _():
        imp_ref[...] = jnp.zeros_like(imp_ref)
        cnt_ref[...] = jnp.zeros_like(cnt_ref)

    imp_ref[...] += jnp.sum(probs, axis=0, keepdims=True)
    cnt_ref[...] += jnp.sum((gates > 0.0).astype(jnp.float32),
                            axis=0, keepdims=True)


def _moe_kernel(xn_ref, o1_ref, gates_ref, w1_ref, b1_ref, w2_ref, b2_ref,
                out_ref):
    e = pl.program_id(0)
    tb = pl.program_id(1)
    xn = xn_ref[...]
    h = jax.nn.gelu(jax.lax.dot(xn, w1_ref[0], precision=HIGHEST)
                    + b1_ref[0])
    o = jax.lax.dot(h, w2_ref[0], precision=HIGHEST) + b2_ref[0]
    lane = jax.lax.broadcasted_iota(jnp.int32, gates_ref.shape, 1)
    gcol = jnp.sum(jnp.where(lane == e, gates_ref[...], 0.0),
                   axis=-1, keepdims=True)
    contrib = gcol * o
    rows = pl.ds(tb * ROW_BLK, ROW_BLK)

    @pl.when(e == 0)
    def _():
        out_ref[rows, :] = o1_ref[...] + contrib

    @pl.when(e != 0)
    def _():
        out_ref[rows, :] += contrib


def _rope_tables():
    half = D_K // 2
    freqs = 1.0 / (10000.0 ** (np.arange(0, D_K, 2, dtype=np.float32) / D_K))
    ang = np.arange(T, dtype=np.float32)[:, None] * freqs[None, :]
    cos = np.repeat(np.cos(ang), 2, axis=1)
    sin = np.repeat(np.sin(ang), 2, axis=1)
    assert cos.shape == (T, D_K) and half * 2 == D_K
    return (np.tile(cos, (1, N_HEADS)).astype(np.float32),
            np.tile(sin, (1, N_HEADS)).astype(np.float32))


def _head_group_matrix():
    g = np.zeros((D_MODEL, N_HEADS), np.float32)
    for h in range(N_HEADS):
        g[h * D_K:(h + 1) * D_K, h] = 1.0
    return g


@jax.jit
def kernel(x, norm1_w, qkv_w, qnorm_w, knorm_w, wo_w, norm2_w, gate_w,
           w1, b1, w2, b2):
    xf = x.reshape(T, D_MODEL)
    n_blocks = T // ROW_BLK
    cos_np, sin_np = _rope_tables()
    g_np = _head_group_matrix()

    q, k, v = pl.pallas_call(
        _qkv_kernel,
        grid=(n_blocks,),
        in_specs=[
            pl.BlockSpec((ROW_BLK, D_MODEL), lambda i: (i, 0)),
            pl.BlockSpec((1, D_MODEL), lambda i: (0, 0)),
            pl.BlockSpec((D_MODEL, 3 * D_MODEL), lambda i: (0, 0)),
            pl.BlockSpec((1, D_MODEL), lambda i: (0, 0)),
            pl.BlockSpec((1, D_MODEL), lambda i: (0, 0)),
            pl.BlockSpec((ROW_BLK, D_MODEL), lambda i: (i, 0)),
            pl.BlockSpec((ROW_BLK, D_MODEL), lambda i: (i, 0)),
            pl.BlockSpec((D_MODEL, N_HEADS), lambda i: (0, 0)),
            pl.BlockSpec((N_HEADS, D_MODEL), lambda i: (0, 0)),
        ],
        out_specs=[
            pl.BlockSpec((N_HEADS, ROW_BLK, D_K), lambda i: (0, i, 0)),
            pl.BlockSpec((N_HEADS, ROW_BLK, D_K), lambda i: (0, i, 0)),
            pl.BlockSpec((N_HEADS, ROW_BLK, D_K), lambda i: (0, i, 0)),
        ],
        out_shape=[jax.ShapeDtypeStruct((N_HEADS, T, D_K), jnp.float32)] * 3,
    )(xf, norm1_w.reshape(1, D_MODEL), qkv_w,
      jnp.tile(qnorm_w, N_HEADS).reshape(1, D_MODEL),
      jnp.tile(knorm_w, N_HEADS).reshape(1, D_MODEL),
      jnp.asarray(cos_np), jnp.asarray(sin_np),
      jnp.asarray(g_np), jnp.asarray(g_np.T))

    attn = pl.pallas_call(
        _attn_kernel,
        grid=(N_HEADS, n_blocks),
        in_specs=[
            pl.BlockSpec((1, ROW_BLK, D_K), lambda h, i: (h, i, 0)),
            pl.BlockSpec((1, T, D_K), lambda h, i: (h, 0, 0)),
            pl.BlockSpec((1, T, D_K), lambda h, i: (h, 0, 0)),
        ],
        out_specs=pl.BlockSpec((1, ROW_BLK, D_K), lambda h, i: (h, i, 0)),
        out_shape=jax.ShapeDtypeStruct((N_HEADS, T, D_K), jnp.float32),
    )(q, k, v)

    o1, xn2, gates, imp, cnt = pl.pallas_call(
        _proj_router_kernel,
        grid=(n_blocks,),
        in_specs=[
            pl.BlockSpec((ROW_BLK, D_MODEL), lambda i: (i, 0)),
            pl.BlockSpec((N_HEADS, ROW_BLK, D_K), lambda i: (0, i, 0)),
            pl.BlockSpec((D_MODEL, D_MODEL), lambda i: (0, 0)),
            pl.BlockSpec((1, D_MODEL), lambda i: (0, 0)),
            pl.BlockSpec((D_MODEL, NUM_EXPERTS), lambda i: (0, 0)),
        ],
        out_specs=[
            pl.BlockSpec((ROW_BLK, D_MODEL), lambda i: (i, 0)),
            pl.BlockSpec((ROW_BLK, D_MODEL), lambda i: (i, 0)),
            pl.BlockSpec((ROW_BLK, NUM_EXPERTS), lambda i: (i, 0)),
            pl.BlockSpec((1, NUM_EXPERTS), lambda i: (0, 0)),
            pl.BlockSpec((1, NUM_EXPERTS), lambda i: (0, 0)),
        ],
        out_shape=[
            jax.ShapeDtypeStruct((T, D_MODEL), jnp.float32),
            jax.ShapeDtypeStruct((T, D_MODEL), jnp.float32),
            jax.ShapeDtypeStruct((T, NUM_EXPERTS), jnp.float32),
            jax.ShapeDtypeStruct((1, NUM_EXPERTS), jnp.float32),
            jax.ShapeDtypeStruct((1, NUM_EXPERTS), jnp.float32),
        ],
    )(xf, attn, wo_w, norm2_w.reshape(1, D_MODEL), gate_w)

    out = pl.pallas_call(
        _moe_kernel,
        grid=(NUM_EXPERTS, n_blocks),
        in_specs=[
            pl.BlockSpec((ROW_BLK, D_MODEL), lambda e, i: (i, 0)),
            pl.BlockSpec((ROW_BLK, D_MODEL), lambda e, i: (i, 0)),
            pl.BlockSpec((ROW_BLK, NUM_EXPERTS), lambda e, i: (i, 0)),
            pl.BlockSpec((1, D_MODEL, D_FF), lambda e, i: (e, 0, 0)),
            pl.BlockSpec((1, 1, D_FF), lambda e, i: (e, 0, 0)),
            pl.BlockSpec((1, D_FF, D_MODEL), lambda e, i: (e, 0, 0)),
            pl.BlockSpec((1, 1, D_MODEL), lambda e, i: (e, 0, 0)),
        ],
        out_specs=pl.BlockSpec((T, D_MODEL), lambda e, i: (0, 0)),
        out_shape=jax.ShapeDtypeStruct((T, D_MODEL), jnp.float32),
    )(xn2, o1, gates, w1, b1.reshape(NUM_EXPERTS, 1, D_FF), w2,
      b2.reshape(NUM_EXPERTS, 1, D_MODEL))

    n = jnp.float32(T)
    aux = jnp.float32(NUM_EXPERTS) * jnp.sum((cnt[0] / n) * (imp[0] / n))
    return out.reshape(1, T, D_MODEL), aux


# speedup vs baseline: 1.9285x; 1.9285x over previous
import jax
import jax.numpy as jnp
import numpy as np
from jax.experimental import pallas as pl

D_MODEL = 768
N_HEADS = 12
D_K = D_MODEL // N_HEADS
D_FF = 768
NUM_EXPERTS = 8
T = 2048
EPS = float(jnp.finfo(jnp.float32).eps)

ROW_BLK = 256
F32 = jnp.float32
BF16 = jnp.bfloat16


def _split(a):
    hi = a.astype(BF16)
    lo = (a - hi.astype(F32)).astype(BF16)
    return hi, lo


def _dg(a, b, dims=(((1,), (0,)), ((), ()))):
    return jax.lax.dot_general(a, b, dims, preferred_element_type=F32)


def _dot3(ah, al, bh, bl, dims=(((1,), (0,)), ((), ()))):
    out = _dg(ah, bh, dims) + _dg(ah, bl, dims)
    return out + _dg(al, bh, dims)


def _rope_rotate(v):
    left = jnp.roll(v, -1, axis=-1)
    right = jnp.roll(v, 1, axis=-1)
    lane = jax.lax.broadcasted_iota(jnp.int32, v.shape, 1)
    even = (lane % 2) == 0
    return jnp.where(even, -left, right)


def _qkv_kernel(x_ref, n1_ref, wh_ref, wl_ref, qnw_ref, knw_ref,
                cos_ref, sin_ref, g_ref, gt_ref,
                qh_ref, ql_ref, kh_ref, kl_ref, vh_ref, vl_ref):
    xr = x_ref[...]
    ms = jnp.mean(xr * xr, axis=-1, keepdims=True)
    xn = xr / jnp.sqrt(ms + EPS) * n1_ref[...]
    xh, xl = _split(xn)
    qkv = _dot3(xh, xl, wh_ref[...], wl_ref[...])
    q = qkv[:, :D_MODEL]
    k = qkv[:, D_MODEL:2 * D_MODEL]
    v = qkv[:, 2 * D_MODEL:]

    cos = cos_ref[...]
    sin = sin_ref[...]
    g = g_ref[...].astype(BF16)
    gt = gt_ref[...].astype(BF16)

    def headnorm_rope(a, w_head):
        a2h, a2l = _split(a * a)
        ss = (_dg(a2h, g) + _dg(a2l, g)) * (1.0 / D_K)
        inv = 1.0 / jnp.sqrt(ss + EPS)
        ih, il = _split(inv)
        invf = _dg(ih, gt) + _dg(il, gt)
        an = a * invf * w_head
        return an * cos + _rope_rotate(an) * sin

    qo = headnorm_rope(q, qnw_ref[...])
    ko = headnorm_rope(k, knw_ref[...])
    qoh, qol = _split(qo)
    koh, kol = _split(ko)
    vh, vl = _split(v)
    for h in range(N_HEADS):
        sl = slice(h * D_K, (h + 1) * D_K)
        qh_ref[h] = qoh[:, sl]
        ql_ref[h] = qol[:, sl]
        kh_ref[h] = koh[:, sl]
        kl_ref[h] = kol[:, sl]
        vh_ref[h] = vh[:, sl]
        vl_ref[h] = vl[:, sl]


_DIMS_NT = (((1,), (1,)), ((), ()))


def _attn_kernel(qh_ref, ql_ref, kh_ref, kl_ref, vh_ref, vl_ref, o_ref):
    qb = pl.program_id(1)
    s = _dot3(qh_ref[0], ql_ref[0], kh_ref[0], kl_ref[0], _DIMS_NT)
    s = s * (1.0 / float(np.sqrt(D_K)))
    row = qb * ROW_BLK + jax.lax.broadcasted_iota(jnp.int32, s.shape, 0)
    col = jax.lax.broadcasted_iota(jnp.int32, s.shape, 1)
    s = jnp.where(row >= col, s, F32(-1e30))
    m = jnp.max(s, axis=-1, keepdims=True)
    e = jnp.exp(s - m)
    p = e / jnp.sum(e, axis=-1, keepdims=True)
    ph = p.astype(BF16)
    o_ref[0] = _dg(ph, vh_ref[0]) + _dg(ph, vl_ref[0])


def _proj_router_kernel(x_ref, a_ref, woh_ref, wol_ref, n2_ref, gw_ref,
                        o1_ref, xnh_ref, xnl_ref, gates_ref,
                        imp_ref, cnt_ref):
    i = pl.program_id(0)
    a = jnp.concatenate([a_ref[h] for h in range(N_HEADS)], axis=-1)
    ah, al = _split(a)
    o1 = x_ref[...] + _dot3(ah, al, woh_ref[...], wol_ref[...])
    o1_ref[...] = o1
    ms = jnp.mean(o1 * o1, axis=-1, keepdims=True)
    xn = o1 / jnp.sqrt(ms + EPS) * n2_ref[...]
    xnh, xnl = _split(xn)
    xnh_ref[...] = xnh
    xnl_ref[...] = xnl

    gwh, gwl = _split(gw_ref[...])
    logits = _dot3(xnh, xnl, gwh, gwl)
    lm = jnp.max(logits, axis=-1, keepdims=True)
    el = jnp.exp(logits - lm)
    probs = el / jnp.sum(el, axis=-1, keepdims=True)

    lane = jax.lax.broadcasted_iota(jnp.int32, probs.shape, 1)
    m1 = jnp.max(probs, axis=-1, keepdims=True)
    i1 = jnp.min(jnp.where(probs == m1, lane, NUM_EXPERTS),
                 axis=-1, keepdims=True)
    masked = jnp.where(lane == i1, F32(-1.0), probs)
    m2 = jnp.max(masked, axis=-1, keepdims=True)
    i2 = jnp.min(jnp.where(masked == m2, lane, NUM_EXPERTS),
                 axis=-1, keepdims=True)
    denom = m1 + m2
    gates = (jnp.where(lane == i1, m1 / denom, 0.0)
             + jnp.where(lane == i2, m2 / denom, 0.0))
    gates_ref[...] = gates

    @pl.when(i == 0)
    def _():
        imp_ref[...] = jnp.zeros_like(imp_ref)
        cnt_ref[...] = jnp.zeros_like(cnt_ref)

    imp_ref[...] += jnp.sum(probs, axis=0, keepdims=True)
    cnt_ref[...] += jnp.sum((gates > 0.0).astype(F32),
                            axis=0, keepdims=True)


def _moe_kernel(xnh_ref, xnl_ref, o1_ref, gates_ref, w1h_ref, w1l_ref,
                b1_ref, w2h_ref, w2l_ref, b2_ref, out_ref):
    e = pl.program_id(0)
    tb = pl.program_id(1)
    h = jax.nn.gelu(_dot3(xnh_ref[...], xnl_ref[...],
                          w1h_ref[0], w1l_ref[0]) + b1_ref[0])
    hh, hl = _split(h)
    o = _dot3(hh, hl, w2h_ref[0], w2l_ref[0]) + b2_ref[0]
    lane = jax.lax.broadcasted_iota(jnp.int32, gates_ref.shape, 1)
    gcol = jnp.sum(jnp.where(lane == e, gates_ref[...], 0.0),
                   axis=-1, keepdims=True)
    contrib = gcol * o
    rows = pl.ds(tb * ROW_BLK, ROW_BLK)

    @pl.when(e == 0)
    def _():
        out_ref[rows, :] = o1_ref[...] + contrib

    @pl.when(e != 0)
    def _():
        out_ref[rows, :] += contrib


def _rope_tables():
    freqs = 1.0 / (10000.0 ** (np.arange(0, D_K, 2, dtype=np.float32) / D_K))
    ang = np.arange(T, dtype=np.float32)[:, None] * freqs[None, :]
    cos = np.repeat(np.cos(ang), 2, axis=1)
    sin = np.repeat(np.sin(ang), 2, axis=1)
    return (np.tile(cos, (1, N_HEADS)).astype(np.float32),
            np.tile(sin, (1, N_HEADS)).astype(np.float32))


def _head_group_matrix():
    g = np.zeros((D_MODEL, N_HEADS), np.float32)
    for h in range(N_HEADS):
        g[h * D_K:(h + 1) * D_K, h] = 1.0
    return g


@jax.jit
def kernel(x, norm1_w, qkv_w, qnorm_w, knorm_w, wo_w, norm2_w, gate_w,
           w1, b1, w2, b2):
    xf = x.reshape(T, D_MODEL)
    n_blocks = T // ROW_BLK
    cos_np, sin_np = _rope_tables()
    g_np = _head_group_matrix()

    qkv_wh, qkv_wl = _split(qkv_w)
    wo_h, wo_l = _split(wo_w)
    w1h, w1l = _split(w1)
    w2h, w2l = _split(w2)

    hsd = lambda s, d=F32: jax.ShapeDtypeStruct(s, d)
    htd = (N_HEADS, T, D_K)

    qh, ql, kh, kl, vh, vl = pl.pallas_call(
        _qkv_kernel,
        grid=(n_blocks,),
        in_specs=[
            pl.BlockSpec((ROW_BLK, D_MODEL), lambda i: (i, 0)),
            pl.BlockSpec((1, D_MODEL), lambda i: (0, 0)),
            pl.BlockSpec((D_MODEL, 3 * D_MODEL), lambda i: (0, 0)),
            pl.BlockSpec((D_MODEL, 3 * D_MODEL), lambda i: (0, 0)),
            pl.BlockSpec((1, D_MODEL), lambda i: (0, 0)),
            pl.BlockSpec((1, D_MODEL), lambda i: (0, 0)),
            pl.BlockSpec((ROW_BLK, D_MODEL), lambda i: (i, 0)),
            pl.BlockSpec((ROW_BLK, D_MODEL), lambda i: (i, 0)),
            pl.BlockSpec((D_MODEL, N_HEADS), lambda i: (0, 0)),
            pl.BlockSpec((N_HEADS, D_MODEL), lambda i: (0, 0)),
        ],
        out_specs=[
            pl.BlockSpec((N_HEADS, ROW_BLK, D_K), lambda i: (0, i, 0))] * 6,
        out_shape=[hsd(htd, BF16)] * 6,
    )(xf, norm1_w.reshape(1, D_MODEL), qkv_wh, qkv_wl,
      jnp.tile(qnorm_w, N_HEADS).reshape(1, D_MODEL),
      jnp.tile(knorm_w, N_HEADS).reshape(1, D_MODEL),
      jnp.asarray(cos_np), jnp.asarray(sin_np),
      jnp.asarray(g_np), jnp.asarray(g_np.T))

    attn = pl.pallas_call(
        _attn_kernel,
        grid=(N_HEADS, n_blocks),
        in_specs=[
            pl.BlockSpec((1, ROW_BLK, D_K), lambda h, i: (h, i, 0)),
            pl.BlockSpec((1, ROW_BLK, D_K), lambda h, i: (h, i, 0)),
            pl.BlockSpec((1, T, D_K), lambda h, i: (h, 0, 0)),
            pl.BlockSpec((1, T, D_K), lambda h, i: (h, 0, 0)),
            pl.BlockSpec((1, T, D_K), lambda h, i: (h, 0, 0)),
            pl.BlockSpec((1, T, D_K), lambda h, i: (h, 0, 0)),
        ],
        out_specs=pl.BlockSpec((1, ROW_BLK, D_K), lambda h, i: (h, i, 0)),
        out_shape=hsd(htd),
    )(qh, ql, kh, kl, vh, vl)

    o1, xnh, xnl, gates, imp, cnt = pl.pallas_call(
        _proj_router_kernel,
        grid=(n_blocks,),
        in_specs=[
            pl.BlockSpec((ROW_BLK, D_MODEL), lambda i: (i, 0)),
            pl.BlockSpec((N_HEADS, ROW_BLK, D_K), lambda i: (0, i, 0)),
            pl.BlockSpec((D_MODEL, D_MODEL), lambda i: (0, 0)),
            pl.BlockSpec((D_MODEL, D_MODEL), lambda i: (0, 0)),
            pl.BlockSpec((1, D_MODEL), lambda i: (0, 0)),
            pl.BlockSpec((D_MODEL, NUM_EXPERTS), lambda i: (0, 0)),
        ],
        out_specs=[
            pl.BlockSpec((ROW_BLK, D_MODEL), lambda i: (i, 0)),
            pl.BlockSpec((ROW_BLK, D_MODEL), lambda i: (i, 0)),
            pl.BlockSpec((ROW_BLK, D_MODEL), lambda i: (i, 0)),
            pl.BlockSpec((ROW_BLK, NUM_EXPERTS), lambda i: (i, 0)),
            pl.BlockSpec((1, NUM_EXPERTS), lambda i: (0, 0)),
            pl.BlockSpec((1, NUM_EXPERTS), lambda i: (0, 0)),
        ],
        out_shape=[
            hsd((T, D_MODEL)),
            hsd((T, D_MODEL), BF16),
            hsd((T, D_MODEL), BF16),
            hsd((T, NUM_EXPERTS)),
            hsd((1, NUM_EXPERTS)),
            hsd((1, NUM_EXPERTS)),
        ],
    )(xf, attn, wo_h, wo_l, norm2_w.reshape(1, D_MODEL), gate_w)

    out = pl.pallas_call(
        _moe_kernel,
        grid=(NUM_EXPERTS, n_blocks),
        in_specs=[
            pl.BlockSpec((ROW_BLK, D_MODEL), lambda e, i: (i, 0)),
            pl.BlockSpec((ROW_BLK, D_MODEL), lambda e, i: (i, 0)),
            pl.BlockSpec((ROW_BLK, D_MODEL), lambda e, i: (i, 0)),
            pl.BlockSpec((ROW_BLK, NUM_EXPERTS), lambda e, i: (i, 0)),
            pl.BlockSpec((1, D_MODEL, D_FF), lambda e, i: (e, 0, 0)),
            pl.BlockSpec((1, D_MODEL, D_FF), lambda e, i: (e, 0, 0)),
            pl.BlockSpec((1, 1, D_FF), lambda e, i: (e, 0, 0)),
            pl.BlockSpec((1, D_FF, D_MODEL), lambda e, i: (e, 0, 0)),
            pl.BlockSpec((1, D_FF, D_MODEL), lambda e, i: (e, 0, 0)),
            pl.BlockSpec((1, 1, D_MODEL), lambda e, i: (e, 0, 0)),
        ],
        out_specs=pl.BlockSpec((T, D_MODEL), lambda e, i: (0, 0)),
        out_shape=hsd((T, D_MODEL)),
    )(xnh, xnl, o1, gates, w1h, w1l, b1.reshape(NUM_EXPERTS, 1, D_FF),
      w2h, w2l, b2.reshape(NUM_EXPERTS, 1, D_MODEL))

    n = F32(T)
    aux = F32(NUM_EXPERTS) * jnp.sum((cnt[0] / n) * (imp[0] / n))
    return out.reshape(1, T, D_MODEL), aux
